# Initial kernel scaffold; baseline (speedup 1.0000x reference)
import jax
import jax.numpy as jnp
from jax.experimental import pallas as pl


def kernel(action, src_node_features, neighbor_node_features):
    raise NotImplementedError("write your pallas kernel here")


# speedup vs baseline: 52.8465x; 52.8465x over previous
import functools

import jax
import jax.numpy as jnp
from jax import lax
from jax.experimental import pallas as pl
from jax.experimental.pallas import tpu as pltpu
from jax.experimental.pallas import tpu_sc as plsc

N_NODES = 10000
D_FEAT = 256
M_ROWS = 160000

NUM_WORKERS = 32
NODES_PER_WORKER = 320
G = 16
RMAX = 16 * G + 32
LANES = 16
J = D_FEAT // LANES


def _hsum16(v):
  s = v[0]
  for i in range(1, LANES):
    s = s + v[i]
  return s


def _worker_body(act_hbm, src_hbm, nbr_hbm, out_hbm, act_v, rows_v, src_v, out_v):
  cid = lax.axis_index("c")
  sid = lax.axis_index("s")
  wid = sid * 2 + cid
  base = wid * NODES_PER_WORKER
  n_nodes = jnp.minimum(NODES_PER_WORKER, N_NODES - base)
  n_groups = n_nodes // G

  pltpu.sync_copy(act_hbm, act_v)

  def _chunk_sum(i, acc):
    return acc + act_v[pl.ds(i * LANES, LANES)]

  acc0 = lax.fori_loop(0, base // LANES, _chunk_sum,
                       jnp.zeros((LANES,), jnp.int32))
  row0 = base + _hsum16(acc0)

  def _group_body(g, row_off):
    nbase = base + g * G
    act_vec = act_v[pl.ds(nbase, LANES)]
    cnt_vec = act_vec + 1
    inv_vec = 1.0 / cnt_vec.astype(jnp.float32)
    gs = G + _hsum16(act_vec)
    a = (row_off // 8) * 8
    need = row_off + gs - a
    nds = (need + 15) // 16
    dstart = jnp.minimum(a, M_ROWS - nds * 16)
    delta = row_off - dstart

    def _dma_chunk(d, carry):
      pltpu.sync_copy(nbr_hbm.at[pl.ds(dstart + d * 16, 16)],
                      rows_v.at[pl.ds(d * 16, 16)])
      return carry

    lax.fori_loop(0, nds, _dma_chunk, 0)
    pltpu.sync_copy(src_hbm.at[pl.ds(nbase, G)], src_v)

    pos = delta
    for i in range(G):
      cnt = cnt_vec[i]
      inv = jnp.broadcast_to(inv_vec[i], (LANES,))

      def _row_body(k, accs, pos=pos):
        r = pos + k
        return tuple(a + rows_v[r, pl.ds(j * LANES, LANES)]
                     for j, a in enumerate(accs))

      accs = lax.fori_loop(
          0, cnt, _row_body,
          tuple(jnp.zeros((LANES,), jnp.float32) for _ in range(J)))
      for j in range(J):
        h = (src_v[i, pl.ds(j * LANES, LANES)] + accs[j] * inv) * 0.5
        out_v[i, pl.ds(j * LANES, LANES)] = jnp.maximum(h, 0.0)
      pos = pos + cnt

    pltpu.sync_copy(out_v, out_hbm.at[pl.ds(nbase, G)])
    return row_off + gs

  lax.fori_loop(0, n_groups, _group_body, row0)


@jax.jit
def kernel(action, src_node_features, neighbor_node_features):
  mesh = plsc.VectorSubcoreMesh(core_axis_name="c", subcore_axis_name="s")
  run = functools.partial(
      pl.kernel,
      out_type=jax.ShapeDtypeStruct((N_NODES, D_FEAT), jnp.float32),
      mesh=mesh,
      scratch_types=[
          pltpu.VMEM((N_NODES,), jnp.int32),
          pltpu.VMEM((RMAX, D_FEAT), jnp.float32),
          pltpu.VMEM((G, D_FEAT), jnp.float32),
          pltpu.VMEM((G, D_FEAT), jnp.float32),
      ],
  )(_worker_body)
  return run(action, src_node_features, neighbor_node_features)


# speedup vs baseline: 126.8627x; 2.4006x over previous
import functools

import jax
import jax.numpy as jnp
from jax import lax
from jax.experimental import pallas as pl
from jax.experimental.pallas import tpu as pltpu
from jax.experimental.pallas import tpu_sc as plsc

N_NODES = 10000
D_FEAT = 256
M_ROWS = 160000

NUM_WORKERS = 32
NODES_PER_WORKER = 320
G = 8
RMAX = 144
LANES = 16
J = D_FEAT // LANES
ACT_PAD = N_NODES + 16


def _hsum8(v, off):
  s = v[off]
  for i in range(1, 8):
    s = s + v[off + i]
  return s


def _row_window(row_off, gs):
  a = (row_off // 8) * 8
  need = row_off + gs - a
  nds = (need + 15) // 16
  dstart = jnp.minimum(a, M_ROWS - nds * 16)
  delta = row_off - dstart
  return dstart, delta, nds


def _worker_body(act_hbm, src_hbm, nbr_hbm, out_hbm, act_v,
                 rows0, rows1, src0, src1, out0, out1,
                 semr0, semr1, sems0, sems1, semo0, semo1):
  cid = lax.axis_index("c")
  sid = lax.axis_index("s")
  wid = sid * 2 + cid
  base = wid * NODES_PER_WORKER
  n_nodes = jnp.minimum(NODES_PER_WORKER, N_NODES - base)
  n_pairs = n_nodes // (2 * G)

  pltpu.sync_copy(act_hbm, act_v.at[pl.ds(0, N_NODES)])

  def _chunk_sum(i, acc):
    return acc + act_v[pl.ds(i * LANES, LANES)]

  acc0 = lax.fori_loop(0, base // LANES, _chunk_sum,
                       jnp.zeros((LANES,), jnp.int32))
  s = acc0[0]
  for i in range(1, LANES):
    s = s + acc0[i]
  row_off0 = base + s

  def _fire_rows(dstart, nds, rows_v, sem):
    def _f(d, c):
      pltpu.async_copy(nbr_hbm.at[pl.ds(dstart + d * 16, 16)],
                       rows_v.at[pl.ds(d * 16, 16)], sem)
      return c
    lax.fori_loop(0, nds, _f, 0)

  def _drain_rows(nds, rows_v, sem):
    def _f(d, c):
      pltpu.make_async_copy(nbr_hbm.at[pl.ds(0, 16)],
                            rows_v.at[pl.ds(0, 16)], sem).wait()
      return c
    lax.fori_loop(0, nds, _f, 0)

  def _compute_group(chunk, inv16, lane_off, delta, rows_v, src_v, out_v):
    pos = delta
    for i in range(G):
      cnt = chunk[lane_off + i] + 1
      inv = jnp.broadcast_to(inv16[lane_off + i], (LANES,))

      def _row_body(k, accs, pos=pos):
        r = pos + k
        return tuple(a + rows_v[r, pl.ds(j * LANES, LANES)]
                     for j, a in enumerate(accs))

      accs = lax.fori_loop(
          0, cnt, _row_body,
          tuple(jnp.zeros((LANES,), jnp.float32) for _ in range(J)))
      for j in range(J):
        h = (src_v[i, pl.ds(j * LANES, LANES)] + accs[j] * inv) * 0.5
        out_v[i, pl.ds(j * LANES, LANES)] = jnp.maximum(h, 0.0)
      pos = pos + cnt

  chunk0 = act_v[pl.ds(base, LANES)]
  gs0 = G + _hsum8(chunk0, 0)
  gs1 = G + _hsum8(chunk0, 8)
  ds0, d0, n0 = _row_window(row_off0, gs0)
  _fire_rows(ds0, n0, rows0, semr0)
  pltpu.async_copy(src_hbm.at[pl.ds(base, G)], src0, sems0)
  ds1, d1, n1 = _row_window(row_off0 + gs0, gs1)
  _fire_rows(ds1, n1, rows1, semr1)
  pltpu.async_copy(src_hbm.at[pl.ds(base + G, G)], src1, sems1)

  def _pair_body(t, carry):
    chunk, d0, n0, d1, n1, row_off2 = carry
    cnt16 = chunk + 1
    inv16 = 1.0 / cnt16.astype(jnp.float32)
    chunk_n = act_v[pl.ds(base + (t + 1) * LANES, LANES)]
    gs2 = G + _hsum8(chunk_n, 0)
    gs3 = G + _hsum8(chunk_n, 8)
    valid = t + 1 < n_pairs

    nbase0 = base + t * 2 * G
    pltpu.make_async_copy(src_hbm.at[pl.ds(0, G)], src0, sems0).wait()
    _drain_rows(n0, rows0, semr0)

    @pl.when(t > 0)
    def _():
      pltpu.make_async_copy(out0, out_hbm.at[pl.ds(0, G)], semo0).wait()

    _compute_group(chunk, inv16, 0, d0, rows0, src0, out0)
    pltpu.async_copy(out0, out_hbm.at[pl.ds(nbase0, G)], semo0)

    ds2, d2, n2 = _row_window(row_off2, gs2)

    @pl.when(valid)
    def _():
      _fire_rows(ds2, n2, rows0, semr0)
      pltpu.async_copy(src_hbm.at[pl.ds(nbase0 + 2 * G, G)], src0, sems0)

    nbase1 = nbase0 + G
    pltpu.make_async_copy(src_hbm.at[pl.ds(0, G)], src1, sems1).wait()
    _drain_rows(n1, rows1, semr1)

    @pl.when(t > 0)
    def _():
      pltpu.make_async_copy(out1, out_hbm.at[pl.ds(0, G)], semo1).wait()

    _compute_group(chunk, inv16, 8, d1, rows1, src1, out1)
    pltpu.async_copy(out1, out_hbm.at[pl.ds(nbase1, G)], semo1)

    row_off3 = row_off2 + gs2
    ds3, d3, n3 = _row_window(row_off3, gs3)

    @pl.when(valid)
    def _():
      _fire_rows(ds3, n3, rows1, semr1)
      pltpu.async_copy(src_hbm.at[pl.ds(nbase1 + 2 * G, G)], src1, sems1)

    return (chunk_n, d2, n2, d3, n3, row_off3 + gs3)

  lax.fori_loop(0, n_pairs, _pair_body,
                (chunk0, d0, n0, d1, n1, row_off0 + gs0 + gs1))

  pltpu.make_async_copy(out0, out_hbm.at[pl.ds(0, G)], semo0).wait()
  pltpu.make_async_copy(out1, out_hbm.at[pl.ds(0, G)], semo1).wait()


@jax.jit
def kernel(action, src_node_features, neighbor_node_features):
  mesh = plsc.VectorSubcoreMesh(core_axis_name="c", subcore_axis_name="s")
  run = functools.partial(
      pl.kernel,
      out_type=jax.ShapeDtypeStruct((N_NODES, D_FEAT), jnp.float32),
      mesh=mesh,
      scratch_types=[
          pltpu.VMEM((ACT_PAD,), jnp.int32),
          pltpu.VMEM((RMAX, D_FEAT), jnp.float32),
          pltpu.VMEM((RMAX, D_FEAT), jnp.float32),
          pltpu.VMEM((G, D_FEAT), jnp.float32),
          pltpu.VMEM((G, D_FEAT), jnp.float32),
          pltpu.VMEM((G, D_FEAT), jnp.float32),
          pltpu.VMEM((G, D_FEAT), jnp.float32),
          pltpu.SemaphoreType.DMA,
          pltpu.SemaphoreType.DMA,
          pltpu.SemaphoreType.DMA,
          pltpu.SemaphoreType.DMA,
          pltpu.SemaphoreType.DMA,
          pltpu.SemaphoreType.DMA,
      ],
  )(_worker_body)
  return run(action, src_node_features, neighbor_node_features)
